# baseline (speedup 1.0000x reference)
import functools

import jax
import jax.numpy as jnp
from jax import lax
from jax.experimental import pallas as pl
from jax.experimental.pallas import tpu as pltpu
from jax.experimental.pallas import tpu_sc as plsc

_BATCH = 16384
_N = 100000
_D = 64
_MARGIN = 10.0
_NC = 2
_NS = 16
_NW = _NC * _NS
_ROWS_PER_W = _BATCH // _NW
_CHUNK = 128
_NCHUNK = _ROWS_PER_W // _CHUNK

_MV_ROWS = 1024
_MV_GRID = -(-_N // _MV_ROWS)
_PN = _MV_GRID * _MV_ROWS


def _mv_body(pos_b, neg_b, u_b, p_b, n_b):
    u = u_b[0, :]
    p_b[...] = jnp.sum(pos_b[...] * u[None, :], axis=1).reshape(8, 128)
    n_b[...] = jnp.sum(neg_b[...] * u[None, :], axis=1).reshape(8, 128)


@jax.jit
def _tc_matvec(pos, neg, u):
    return pl.pallas_call(
        _mv_body,
        grid=(_MV_GRID,),
        in_specs=[
            pl.BlockSpec((_MV_ROWS, _D), lambda i: (i, 0)),
            pl.BlockSpec((_MV_ROWS, _D), lambda i: (i, 0)),
            pl.BlockSpec((1, _D), lambda i: (0, 0)),
        ],
        out_specs=[
            pl.BlockSpec((8, 128), lambda i: (i, 0)),
            pl.BlockSpec((8, 128), lambda i: (i, 0)),
        ],
        out_shape=[
            jax.ShapeDtypeStruct((_PN // 128, 128), jnp.float32),
            jax.ShapeDtypeStruct((_PN // 128, 128), jnp.float32),
        ],
    )(pos, neg, u)


def _sc_body(idx_hbm, p_hbm, n_hbm, out_hbm,
             idx_v, p_v, n_v, acc_v, sem0, sem1, sem2, sem3):
    sems = [sem0, sem1, sem2, sem3]
    wid = lax.axis_index("s") * _NC + lax.axis_index("c")

    pltpu.sync_copy(idx_hbm.at[pl.ds(wid * _NCHUNK, _NCHUNK)], idx_v)

    descs = []
    for j in range(_NCHUNK):
        dst = pl.ds(j * _CHUNK, _CHUNK)
        descs.append(pltpu.async_copy(p_hbm.at[idx_v.at[j]], p_v.at[dst],
                                      sems[j]))
        descs.append(pltpu.async_copy(n_hbm.at[idx_v.at[j]], n_v.at[dst],
                                      sems[j]))

    acc = jnp.zeros((16,), jnp.float32)
    for j in range(_NCHUNK):
        descs[2 * j].wait()
        descs[2 * j + 1].wait()

        def grp_body(g, acc):
            p16 = p_v[pl.ds(g * 16, 16)]
            n16 = n_v[pl.ds(g * 16, 16)]
            return acc + jnp.maximum(0.0, _MARGIN - p16 + n16)

        acc = plsc.parallel_loop(j * (_CHUNK // 16), (j + 1) * (_CHUNK // 16),
                                 unroll=8, carry=acc)(grp_body)

    acc_v[...] = acc
    pltpu.sync_copy(acc_v, out_hbm.at[wid])


@jax.jit
def _sc_loss_partials(idx2d, pflat, nflat):
    mesh = plsc.VectorSubcoreMesh(core_axis_name="c", subcore_axis_name="s")
    f = pl.kernel(
        _sc_body,
        out_type=jax.ShapeDtypeStruct((_NW, 16), jnp.float32),
        mesh=mesh,
        scratch_types=[
            pltpu.VMEM((_NCHUNK, _CHUNK), jnp.int32),
            pltpu.VMEM((_ROWS_PER_W,), jnp.float32),
            pltpu.VMEM((_ROWS_PER_W,), jnp.float32),
            pltpu.VMEM((16,), jnp.float32),
            pltpu.SemaphoreType.DMA,
            pltpu.SemaphoreType.DMA,
            pltpu.SemaphoreType.DMA,
            pltpu.SemaphoreType.DMA,
        ],
        compiler_params=pltpu.CompilerParams(use_tc_tiling_on_sc=False),
    )
    return f(idx2d, pflat, nflat)


def kernel(idxs, positive_samples, negative_samples, U):
    idx2d = idxs.reshape(_NW * _NCHUNK, _CHUNK).astype(jnp.int32)
    p, n = _tc_matvec(positive_samples, negative_samples, U)
    partials = _sc_loss_partials(idx2d, p.reshape(-1), n.reshape(-1))
    return jnp.sum(partials) / _BATCH


# speedup vs baseline: 4.4673x; 4.4673x over previous
import functools

import jax
import jax.numpy as jnp
from jax import lax
from jax.experimental import pallas as pl
from jax.experimental.pallas import tpu as pltpu
from jax.experimental.pallas import tpu_sc as plsc

_BATCH = 16384
_N = 100000
_D = 64
_MARGIN = 10.0
_NC = 2
_NS = 16
_NW = _NC * _NS
_ROWS_PER_W = _BATCH // _NW
_CHUNK = 128
_NCHUNK = _ROWS_PER_W // _CHUNK

_MV_ROWS = 16384
_MV_GRID = -(-_N // _MV_ROWS)
_PN = _MV_GRID * _MV_ROWS


def _mv_body(pos_b, neg_b, u_b, p_b, n_b):
    dn = (((1,), (0,)), ((), ()))
    u = u_b[...]
    p = jax.lax.dot_general(u, pos_b[...], dn,
                            preferred_element_type=jnp.float32)
    n = jax.lax.dot_general(u, neg_b[...], dn,
                            preferred_element_type=jnp.float32)
    p_b[...] = p.reshape(_MV_ROWS // 128, 128)
    n_b[...] = n.reshape(_MV_ROWS // 128, 128)


@jax.jit
def _tc_matvec(pos_t, neg_t, u):
    return pl.pallas_call(
        _mv_body,
        grid=(_MV_GRID,),
        in_specs=[
            pl.BlockSpec((_D, _MV_ROWS), lambda i: (0, i)),
            pl.BlockSpec((_D, _MV_ROWS), lambda i: (0, i)),
            pl.BlockSpec((1, _D), lambda i: (0, 0)),
        ],
        out_specs=[
            pl.BlockSpec((_MV_ROWS // 128, 128), lambda i: (i, 0)),
            pl.BlockSpec((_MV_ROWS // 128, 128), lambda i: (i, 0)),
        ],
        out_shape=[
            jax.ShapeDtypeStruct((_PN // 128, 128), jnp.float32),
            jax.ShapeDtypeStruct((_PN // 128, 128), jnp.float32),
        ],
    )(pos_t, neg_t, u)


def _sc_body(idx_hbm, p_hbm, n_hbm, out_hbm,
             idx_v, p_v, n_v, acc_v, sem0, sem1, sem2, sem3):
    sems = [sem0, sem1, sem2, sem3]
    wid = lax.axis_index("s") * _NC + lax.axis_index("c")

    pltpu.sync_copy(idx_hbm.at[pl.ds(wid * _NCHUNK, _NCHUNK)], idx_v)

    descs = []
    for j in range(_NCHUNK):
        dst = pl.ds(j * _CHUNK, _CHUNK)
        descs.append(pltpu.async_copy(p_hbm.at[idx_v.at[j]], p_v.at[dst],
                                      sems[j]))
        descs.append(pltpu.async_copy(n_hbm.at[idx_v.at[j]], n_v.at[dst],
                                      sems[j]))

    acc = jnp.zeros((16,), jnp.float32)
    for j in range(_NCHUNK):
        descs[2 * j].wait()
        descs[2 * j + 1].wait()

        def grp_body(g, acc):
            p16 = p_v[pl.ds(g * 16, 16)]
            n16 = n_v[pl.ds(g * 16, 16)]
            return acc + jnp.maximum(0.0, _MARGIN - p16 + n16)

        acc = plsc.parallel_loop(j * (_CHUNK // 16), (j + 1) * (_CHUNK // 16),
                                 unroll=8, carry=acc)(grp_body)

    acc_v[...] = acc
    pltpu.sync_copy(acc_v, out_hbm.at[wid])


@jax.jit
def _sc_loss_partials(idx2d, pflat, nflat):
    mesh = plsc.VectorSubcoreMesh(core_axis_name="c", subcore_axis_name="s")
    f = pl.kernel(
        _sc_body,
        out_type=jax.ShapeDtypeStruct((_NW, 16), jnp.float32),
        mesh=mesh,
        scratch_types=[
            pltpu.VMEM((_NCHUNK, _CHUNK), jnp.int32),
            pltpu.VMEM((_ROWS_PER_W,), jnp.float32),
            pltpu.VMEM((_ROWS_PER_W,), jnp.float32),
            pltpu.VMEM((16,), jnp.float32),
            pltpu.SemaphoreType.DMA,
            pltpu.SemaphoreType.DMA,
            pltpu.SemaphoreType.DMA,
            pltpu.SemaphoreType.DMA,
        ],
        compiler_params=pltpu.CompilerParams(use_tc_tiling_on_sc=False),
    )
    return f(idx2d, pflat, nflat)


def kernel(idxs, positive_samples, negative_samples, U):
    idx2d = idxs.reshape(_NW * _NCHUNK, _CHUNK).astype(jnp.int32)
    p, n = _tc_matvec(positive_samples.T, negative_samples.T, U)
    partials = _sc_loss_partials(idx2d, p.reshape(-1), n.reshape(-1))
    return jnp.sum(partials) / _BATCH
